# baseline (speedup 1.0000x reference)
import functools

import jax
import jax.numpy as jnp
from jax import lax
from jax.experimental import pallas as pl
from jax.experimental.pallas import tpu as pltpu
from jax.experimental.pallas import tpu_sc as plsc

NUM_FEATS = 11
NUM_CATS = 21
HIDDEN = 128
LANES = 16
HWORDS = HIDDEN // 2

_info = plsc.get_sparse_core_info()
NC, NS = _info.num_cores, _info.num_subcores
NW = NC * NS

NPAIR = 3
NSINGLE = NUM_FEATS - 2 * NPAIR
PAIR_ROWS = NUM_CATS * NUM_CATS
TAB_ROWS = NPAIR * PAIR_ROWS + NSINGLE * NUM_CATS
TAB_WORDS = TAB_ROWS * HWORDS
SINGLE_BASE = NPAIR * PAIR_ROWS

N = 100000
ROWS_PER_W = N // NW
CHUNK = 125
NCHUNKS = ROWS_PER_W // CHUNK
OUT_WORDS = CHUNK * HIDDEN
XWIN = 1408


def _prep_body(emb_ref, out_ref):
    for p in range(NPAIR):
        a = emb_ref[2 * p]
        b = emb_ref[2 * p + 1]
        for ca in range(NUM_CATS):
            rows = a[ca:ca + 1, :] + b
            out_ref[pl.ds(p * PAIR_ROWS + ca * NUM_CATS, NUM_CATS), :] = (
                rows.astype(jnp.bfloat16))
    for s in range(NSINGLE):
        out_ref[pl.ds(SINGLE_BASE + s * NUM_CATS, NUM_CATS), :] = (
            emb_ref[2 * NPAIR + s].astype(jnp.bfloat16))


_tc_prep = pl.pallas_call(
    _prep_body,
    out_shape=jax.ShapeDtypeStruct((TAB_ROWS, HIDDEN), jnp.bfloat16),
    in_specs=[pl.BlockSpec(memory_space=pltpu.VMEM)],
    out_specs=pl.BlockSpec(memory_space=pltpu.VMEM),
)


def _compute_chunk(tab_v, xc_v, outc_v, xoff):
    @plsc.parallel_loop(0, CHUNK, unroll=25)
    def row_body(r):
        catv = xc_v[pl.ds(xoff + r * NUM_FEATS, LANES)]
        starts = []
        for p in range(NPAIR):
            ca = catv[2 * p]
            cb = catv[2 * p + 1]
            starts.append((p * PAIR_ROWS + ca * NUM_CATS + cb) * HWORDS)
        for s in range(NSINGLE):
            c = catv[2 * NPAIR + s]
            starts.append((SINGLE_BASE + s * NUM_CATS + c) * HWORDS)
        for hg in range(HWORDS // LANES):
            acc = plsc.bitcast(
                tab_v[pl.ds(starts[0] + hg * LANES, LANES)], jnp.bfloat16)
            for j in range(1, NPAIR + NSINGLE):
                acc = acc + plsc.bitcast(
                    tab_v[pl.ds(starts[j] + hg * LANES, LANES)], jnp.bfloat16)
            lo, hi = plsc.unpack(acc, format=plsc.PackFormat.INTERLEAVED)
            outc_v[pl.ds(r * HIDDEN + hg * 2 * LANES, LANES)] = lo
            outc_v[pl.ds(r * HIDDEN + hg * 2 * LANES + LANES, LANES)] = hi


def _body(xf_hbm, tabf_hbm, outf_hbm, tab_v, xc_v, outc0, outc1, sem0, sem1):
    wid = lax.axis_index("s") * NC + lax.axis_index("c")
    base = wid * ROWS_PER_W
    outcs = (outc0, outc1)
    sems = (sem0, sem1)

    pltpu.sync_copy(tabf_hbm, tab_v)

    for b in range(2):
        pltpu.make_async_copy(
            outf_hbm.at[pl.ds(0, OUT_WORDS)], outcs[b], sems[b]).start()

    def chunk_step(c, b):
        row0 = base + c * CHUNK
        out_slice = outf_hbm.at[pl.ds(row0 * HIDDEN, OUT_WORDS)]
        w0 = row0 * NUM_FEATS
        a0 = jnp.minimum((w0 // 8) * 8, N * NUM_FEATS - XWIN)
        pltpu.sync_copy(xf_hbm.at[pl.ds(a0, XWIN)],
                        xc_v.at[pl.ds(0, XWIN)])
        pltpu.make_async_copy(outcs[b], out_slice, sems[b]).wait()
        _compute_chunk(tab_v, xc_v, outcs[b], w0 - a0)
        pltpu.make_async_copy(outcs[b], out_slice, sems[b]).start()

    def pair_body(ii, carry):
        for b in range(2):
            chunk_step(ii * 2 + b, b)
        return carry

    lax.fori_loop(0, NCHUNKS // 2, pair_body, 0)
    if NCHUNKS % 2:
        chunk_step(NCHUNKS - 1, 0)

    for b in range(2):
        c = NCHUNKS - 2 + b if NCHUNKS % 2 == 0 else NCHUNKS - 1 - b
        row0 = base + c * CHUNK
        pltpu.make_async_copy(
            outcs[b], outf_hbm.at[pl.ds(row0 * HIDDEN, OUT_WORDS)],
            sems[b]).wait()


@functools.partial(
    pl.kernel,
    mesh=plsc.VectorSubcoreMesh(core_axis_name="c", subcore_axis_name="s"),
    compiler_params=pltpu.CompilerParams(needs_layout_passes=False),
    out_type=jax.ShapeDtypeStruct((N * HIDDEN,), jnp.float32),
    scratch_types=[
        pltpu.VMEM((TAB_WORDS,), jnp.int32),
        pltpu.VMEM((XWIN + LANES,), jnp.int32),
        pltpu.VMEM((OUT_WORDS,), jnp.float32),
        pltpu.VMEM((OUT_WORDS,), jnp.float32),
        pltpu.SemaphoreType.DMA,
        pltpu.SemaphoreType.DMA,
    ],
)
def _sc_encode(xf_hbm, tabf_hbm, outf_hbm, tab_v, xc_v, outc0, outc1,
               sem0, sem1):
    _body(xf_hbm, tabf_hbm, outf_hbm, tab_v, xc_v, outc0, outc1, sem0, sem1)


def kernel(x, emb):
    n = x.shape[0]
    tabbf = _tc_prep(emb)
    tabp = tabbf.reshape(TAB_ROWS, 4, 2, LANES).transpose(0, 1, 3, 2)
    tab_i32 = jax.lax.bitcast_convert_type(
        tabp.reshape(TAB_ROWS * HWORDS, 2), jnp.int32)
    outf = _sc_encode(x.astype(jnp.int32).reshape(-1), tab_i32)
    return outf.reshape(n, HIDDEN)


# speedup vs baseline: 1.0402x; 1.0402x over previous
import functools

import jax
import jax.numpy as jnp
from jax import lax
from jax.experimental import pallas as pl
from jax.experimental.pallas import tpu as pltpu
from jax.experimental.pallas import tpu_sc as plsc

NUM_FEATS = 11
NUM_CATS = 21
HIDDEN = 128
LANES = 16
HWORDS = HIDDEN // 2

_info = plsc.get_sparse_core_info()
NC, NS = _info.num_cores, _info.num_subcores
NW = NC * NS

NPAIR = 3
NSINGLE = NUM_FEATS - 2 * NPAIR
PAIR_ROWS = NUM_CATS * NUM_CATS
TAB_ROWS = NPAIR * PAIR_ROWS + NSINGLE * NUM_CATS
TAB_WORDS = TAB_ROWS * HWORDS
SINGLE_BASE = NPAIR * PAIR_ROWS

N = 100000
ROWS_PER_W = N // NW
CHUNK = 125
NCHUNKS = ROWS_PER_W // CHUNK
OUT_WORDS = CHUNK * HIDDEN
XWIN = 1408


def _prep_body(emb_ref, out_ref):
    for p in range(NPAIR):
        a = emb_ref[2 * p]
        b = emb_ref[2 * p + 1]
        for ca in range(NUM_CATS):
            rows = a[ca:ca + 1, :] + b
            out_ref[pl.ds(p * PAIR_ROWS + ca * NUM_CATS, NUM_CATS), :] = (
                rows.astype(jnp.bfloat16))
    for s in range(NSINGLE):
        out_ref[pl.ds(SINGLE_BASE + s * NUM_CATS, NUM_CATS), :] = (
            emb_ref[2 * NPAIR + s].astype(jnp.bfloat16))


_tc_prep = pl.pallas_call(
    _prep_body,
    out_shape=jax.ShapeDtypeStruct((TAB_ROWS, HIDDEN), jnp.bfloat16),
    in_specs=[pl.BlockSpec(memory_space=pltpu.VMEM)],
    out_specs=pl.BlockSpec(memory_space=pltpu.VMEM),
)


def _compute_chunk(tab_v, xc_v, outc_v, xoff):
    @plsc.parallel_loop(0, CHUNK, unroll=10)
    def row_body(r):
        catv = xc_v[pl.ds(xoff + r * NUM_FEATS, LANES)]
        starts = []
        for p in range(NPAIR):
            ca = catv[2 * p]
            cb = catv[2 * p + 1]
            starts.append((p * PAIR_ROWS + ca * NUM_CATS + cb) * HWORDS)
        for s in range(NSINGLE):
            c = catv[2 * NPAIR + s]
            starts.append((SINGLE_BASE + s * NUM_CATS + c) * HWORDS)
        for hg in range(HWORDS // LANES):
            acc = plsc.bitcast(
                tab_v[pl.ds(starts[0] + hg * LANES, LANES)], jnp.bfloat16)
            for j in range(1, NPAIR + NSINGLE):
                acc = acc + plsc.bitcast(
                    tab_v[pl.ds(starts[j] + hg * LANES, LANES)], jnp.bfloat16)
            lo, hi = plsc.unpack(acc, format=plsc.PackFormat.INTERLEAVED)
            outc_v[pl.ds(r * HIDDEN + hg * 2 * LANES, LANES)] = lo
            outc_v[pl.ds(r * HIDDEN + hg * 2 * LANES + LANES, LANES)] = hi


def _body(xf_hbm, tabf_hbm, outf_hbm, tab_v, xc_v, outc0, outc1, sem0, sem1):
    wid = lax.axis_index("s") * NC + lax.axis_index("c")
    base = wid * ROWS_PER_W
    outcs = (outc0, outc1)
    sems = (sem0, sem1)

    pltpu.sync_copy(tabf_hbm, tab_v)

    for b in range(2):
        pltpu.make_async_copy(
            outf_hbm.at[pl.ds(0, OUT_WORDS)], outcs[b], sems[b]).start()

    def chunk_step(c, b):
        row0 = base + c * CHUNK
        out_slice = outf_hbm.at[pl.ds(row0 * HIDDEN, OUT_WORDS)]
        w0 = row0 * NUM_FEATS
        a0 = jnp.minimum((w0 // 8) * 8, N * NUM_FEATS - XWIN)
        pltpu.sync_copy(xf_hbm.at[pl.ds(a0, XWIN)],
                        xc_v.at[pl.ds(0, XWIN)])
        pltpu.make_async_copy(outcs[b], out_slice, sems[b]).wait()
        _compute_chunk(tab_v, xc_v, outcs[b], w0 - a0)
        pltpu.make_async_copy(outcs[b], out_slice, sems[b]).start()

    def pair_body(ii, carry):
        for b in range(2):
            chunk_step(ii * 2 + b, b)
        return carry

    lax.fori_loop(0, NCHUNKS // 2, pair_body, 0)
    if NCHUNKS % 2:
        chunk_step(NCHUNKS - 1, 0)

    for b in range(2):
        c = NCHUNKS - 2 + b if NCHUNKS % 2 == 0 else NCHUNKS - 1 - b
        row0 = base + c * CHUNK
        pltpu.make_async_copy(
            outcs[b], outf_hbm.at[pl.ds(row0 * HIDDEN, OUT_WORDS)],
            sems[b]).wait()


@functools.partial(
    pl.kernel,
    mesh=plsc.VectorSubcoreMesh(core_axis_name="c", subcore_axis_name="s"),
    compiler_params=pltpu.CompilerParams(needs_layout_passes=False),
    out_type=jax.ShapeDtypeStruct((N * HIDDEN,), jnp.float32),
    scratch_types=[
        pltpu.VMEM((TAB_WORDS,), jnp.int32),
        pltpu.VMEM((XWIN + LANES,), jnp.int32),
        pltpu.VMEM((OUT_WORDS,), jnp.float32),
        pltpu.VMEM((OUT_WORDS,), jnp.float32),
        pltpu.SemaphoreType.DMA,
        pltpu.SemaphoreType.DMA,
    ],
)
def _sc_encode(xf_hbm, tabf_hbm, outf_hbm, tab_v, xc_v, outc0, outc1,
               sem0, sem1):
    _body(xf_hbm, tabf_hbm, outf_hbm, tab_v, xc_v, outc0, outc1, sem0, sem1)


def kernel(x, emb):
    n = x.shape[0]
    tabbf = _tc_prep(emb)
    tabp = tabbf.reshape(TAB_ROWS, 4, 2, LANES).transpose(0, 1, 3, 2)
    tab_i32 = jax.lax.bitcast_convert_type(
        tabp.reshape(TAB_ROWS * HWORDS, 2), jnp.int32)
    outf = _sc_encode(x.astype(jnp.int32).reshape(-1), tab_i32)
    return outf.reshape(n, HIDDEN)


# speedup vs baseline: 1.1536x; 1.1091x over previous
import functools

import jax
import jax.numpy as jnp
from jax import lax
from jax.experimental import pallas as pl
from jax.experimental.pallas import tpu as pltpu
from jax.experimental.pallas import tpu_sc as plsc

NUM_FEATS = 11
NUM_CATS = 21
HIDDEN = 128
LANES = 16
HWORDS = HIDDEN // 2

_info = plsc.get_sparse_core_info()
NC, NS = _info.num_cores, _info.num_subcores
NW = NC * NS

NPAIR = 3
NSINGLE = NUM_FEATS - 2 * NPAIR
PAIR_ROWS = NUM_CATS * NUM_CATS
TAB_ROWS = NPAIR * PAIR_ROWS + NSINGLE * NUM_CATS
TAB_WORDS = TAB_ROWS * HWORDS
SINGLE_BASE = NPAIR * PAIR_ROWS

N = 100000
ROWS_PER_W = N // NW
CHUNK = 125
NCHUNKS = ROWS_PER_W // CHUNK
OUT_WORDS = CHUNK * HIDDEN
XWIN = 1408


def _prep_body(emb_ref, out_ref):
    for p in range(NPAIR):
        a = emb_ref[2 * p]
        b = emb_ref[2 * p + 1]
        for ca in range(NUM_CATS):
            rows = a[ca:ca + 1, :] + b
            out_ref[pl.ds(p * PAIR_ROWS + ca * NUM_CATS, NUM_CATS), :] = (
                rows.astype(jnp.bfloat16))
    for s in range(NSINGLE):
        out_ref[pl.ds(SINGLE_BASE + s * NUM_CATS, NUM_CATS), :] = (
            emb_ref[2 * NPAIR + s].astype(jnp.bfloat16))


_tc_prep = pl.pallas_call(
    _prep_body,
    out_shape=jax.ShapeDtypeStruct((TAB_ROWS, HIDDEN), jnp.bfloat16),
    in_specs=[pl.BlockSpec(memory_space=pltpu.VMEM)],
    out_specs=pl.BlockSpec(memory_space=pltpu.VMEM),
)


def _compute_chunk(tab_v, xc_v, outc_v, xoff):
    NHG = HWORDS // LANES

    @plsc.parallel_loop(0, CHUNK, unroll=5)
    def row_body(r):
        catv = xc_v[pl.ds(xoff + r * NUM_FEATS, LANES)]
        accs = [None] * NHG
        for j in range(NPAIR + NSINGLE):
            if j < NPAIR:
                ca = catv[2 * j]
                cb = catv[2 * j + 1]
                start = (j * PAIR_ROWS + ca * NUM_CATS + cb) * HWORDS
            else:
                s = j - NPAIR
                c = catv[2 * NPAIR + s]
                start = (SINGLE_BASE + s * NUM_CATS + c) * HWORDS
            for hg in range(NHG):
                v = plsc.bitcast(
                    tab_v[pl.ds(start + hg * LANES, LANES)], jnp.bfloat16)
                accs[hg] = v if j == 0 else accs[hg] + v
        for hg in range(NHG):
            lo, hi = plsc.unpack(accs[hg], format=plsc.PackFormat.INTERLEAVED)
            outc_v[pl.ds(r * HIDDEN + hg * 2 * LANES, LANES)] = lo
            outc_v[pl.ds(r * HIDDEN + hg * 2 * LANES + LANES, LANES)] = hi


def _body(xf_hbm, tabf_hbm, outf_hbm, tab_v, xc_v, outc0, outc1, sem0, sem1):
    wid = lax.axis_index("s") * NC + lax.axis_index("c")
    base = wid * ROWS_PER_W
    outcs = (outc0, outc1)
    sems = (sem0, sem1)

    pltpu.sync_copy(tabf_hbm, tab_v)

    for b in range(2):
        pltpu.make_async_copy(
            outf_hbm.at[pl.ds(0, OUT_WORDS)], outcs[b], sems[b]).start()

    def chunk_step(c, b):
        row0 = base + c * CHUNK
        out_slice = outf_hbm.at[pl.ds(row0 * HIDDEN, OUT_WORDS)]
        w0 = row0 * NUM_FEATS
        a0 = jnp.minimum((w0 // 8) * 8, N * NUM_FEATS - XWIN)
        pltpu.sync_copy(xf_hbm.at[pl.ds(a0, XWIN)],
                        xc_v.at[pl.ds(0, XWIN)])
        pltpu.make_async_copy(outcs[b], out_slice, sems[b]).wait()
        _compute_chunk(tab_v, xc_v, outcs[b], w0 - a0)
        pltpu.make_async_copy(outcs[b], out_slice, sems[b]).start()

    def pair_body(ii, carry):
        for b in range(2):
            chunk_step(ii * 2 + b, b)
        return carry

    lax.fori_loop(0, NCHUNKS // 2, pair_body, 0)
    if NCHUNKS % 2:
        chunk_step(NCHUNKS - 1, 0)

    for b in range(2):
        c = NCHUNKS - 2 + b if NCHUNKS % 2 == 0 else NCHUNKS - 1 - b
        row0 = base + c * CHUNK
        pltpu.make_async_copy(
            outcs[b], outf_hbm.at[pl.ds(row0 * HIDDEN, OUT_WORDS)],
            sems[b]).wait()


@functools.partial(
    pl.kernel,
    mesh=plsc.VectorSubcoreMesh(core_axis_name="c", subcore_axis_name="s"),
    compiler_params=pltpu.CompilerParams(needs_layout_passes=False),
    out_type=jax.ShapeDtypeStruct((N * HIDDEN,), jnp.float32),
    scratch_types=[
        pltpu.VMEM((TAB_WORDS,), jnp.int32),
        pltpu.VMEM((XWIN + LANES,), jnp.int32),
        pltpu.VMEM((OUT_WORDS,), jnp.float32),
        pltpu.VMEM((OUT_WORDS,), jnp.float32),
        pltpu.SemaphoreType.DMA,
        pltpu.SemaphoreType.DMA,
    ],
)
def _sc_encode(xf_hbm, tabf_hbm, outf_hbm, tab_v, xc_v, outc0, outc1,
               sem0, sem1):
    _body(xf_hbm, tabf_hbm, outf_hbm, tab_v, xc_v, outc0, outc1, sem0, sem1)


def kernel(x, emb):
    n = x.shape[0]
    tabbf = _tc_prep(emb)
    tabp = tabbf.reshape(TAB_ROWS, 4, 2, LANES).transpose(0, 1, 3, 2)
    tab_i32 = jax.lax.bitcast_convert_type(
        tabp.reshape(TAB_ROWS * HWORDS, 2), jnp.int32)
    outf = _sc_encode(x.astype(jnp.int32).reshape(-1), tab_i32)
    return outf.reshape(n, HIDDEN)


# speedup vs baseline: 1.2705x; 1.1013x over previous
import functools

import jax
import jax.numpy as jnp
from jax import lax
from jax.experimental import pallas as pl
from jax.experimental.pallas import tpu as pltpu
from jax.experimental.pallas import tpu_sc as plsc

NUM_FEATS = 11
NUM_CATS = 21
HIDDEN = 128
LANES = 16
HWORDS = HIDDEN // 2

_info = plsc.get_sparse_core_info()
NC, NS = _info.num_cores, _info.num_subcores
NW = NC * NS

NPAIR = 3
NSINGLE = NUM_FEATS - 2 * NPAIR
PAIR_ROWS = NUM_CATS * NUM_CATS
TAB_ROWS = NPAIR * PAIR_ROWS + NSINGLE * NUM_CATS
TAB_WORDS = TAB_ROWS * HWORDS
SINGLE_BASE = NPAIR * PAIR_ROWS

N = 100000
ROWS_PER_W = N // NW
CHUNK = 125
NCHUNKS = ROWS_PER_W // CHUNK
OUT_WORDS = CHUNK * HIDDEN
XWIN = 1408


def _prep_body(emb_ref, out_ref):
    for p in range(NPAIR):
        a = emb_ref[2 * p]
        b = emb_ref[2 * p + 1]
        for ca in range(NUM_CATS):
            rows = a[ca:ca + 1, :] + b
            out_ref[pl.ds(p * PAIR_ROWS + ca * NUM_CATS, NUM_CATS), :] = (
                rows.astype(jnp.bfloat16))
    for s in range(NSINGLE):
        out_ref[pl.ds(SINGLE_BASE + s * NUM_CATS, NUM_CATS), :] = (
            emb_ref[2 * NPAIR + s].astype(jnp.bfloat16))


_tc_prep = pl.pallas_call(
    _prep_body,
    out_shape=jax.ShapeDtypeStruct((TAB_ROWS, HIDDEN), jnp.bfloat16),
    in_specs=[pl.BlockSpec(memory_space=pltpu.VMEM)],
    out_specs=pl.BlockSpec(memory_space=pltpu.VMEM),
)


def _compute_chunk(tab_v, xc_v, outc_v, xoff):
    NHG = HWORDS // LANES

    @plsc.parallel_loop(0, CHUNK, unroll=5)
    def row_body(r):
        catv = xc_v[pl.ds(xoff + r * NUM_FEATS, LANES)]
        accs = [None] * NHG
        for j in range(NPAIR + NSINGLE):
            if j < NPAIR:
                ca = catv[2 * j]
                cb = catv[2 * j + 1]
                start = (j * PAIR_ROWS + ca * NUM_CATS + cb) * HWORDS
            else:
                s = j - NPAIR
                c = catv[2 * NPAIR + s]
                start = (SINGLE_BASE + s * NUM_CATS + c) * HWORDS
            for hg in range(NHG):
                v = plsc.bitcast(
                    tab_v[pl.ds(start + hg * LANES, LANES)], jnp.bfloat16)
                accs[hg] = v if j == 0 else accs[hg] + v
        for hg in range(NHG):
            lo, hi = plsc.unpack(accs[hg], format=plsc.PackFormat.INTERLEAVED)
            outc_v[pl.ds(r * HIDDEN + hg * 2 * LANES, LANES)] = lo
            outc_v[pl.ds(r * HIDDEN + hg * 2 * LANES + LANES, LANES)] = hi


def _body(xf_hbm, tabf_hbm, outf_hbm, tab_v, xc0, xc1, outc0, outc1,
          sem0, sem1, xsem0, xsem1):
    wid = lax.axis_index("s") * NC + lax.axis_index("c")
    base = wid * ROWS_PER_W
    outcs = (outc0, outc1)
    sems = (sem0, sem1)
    xcs = (xc0, xc1)
    xsems = (xsem0, xsem1)

    def x_window(c):
        w0 = (base + c * CHUNK) * NUM_FEATS
        a0 = jnp.minimum((w0 // 8) * 8, N * NUM_FEATS - XWIN)
        return w0, a0

    def x_copy(c, b):
        _, a0 = x_window(c)
        return pltpu.make_async_copy(
            xf_hbm.at[pl.ds(a0, XWIN)], xcs[b].at[pl.ds(0, XWIN)], xsems[b])

    x_copy(0, 0).start()
    pltpu.sync_copy(tabf_hbm, tab_v)

    for b in range(2):
        pltpu.make_async_copy(
            outf_hbm.at[pl.ds(0, OUT_WORDS)], outcs[b], sems[b]).start()

    def chunk_step(c, b, prefetch_next):
        row0 = base + c * CHUNK
        out_slice = outf_hbm.at[pl.ds(row0 * HIDDEN, OUT_WORDS)]
        w0, a0 = x_window(c)
        x_copy(c, b).wait()
        if prefetch_next:
            x_copy(c + 1, 1 - b).start()
        pltpu.make_async_copy(outcs[b], out_slice, sems[b]).wait()
        _compute_chunk(tab_v, xcs[b], outcs[b], w0 - a0)
        pltpu.make_async_copy(outcs[b], out_slice, sems[b]).start()

    def pair_body(ii, carry):
        for b in range(2):
            chunk_step(ii * 2 + b, b, True)
        return carry

    lax.fori_loop(0, NCHUNKS // 2, pair_body, 0)
    chunk_step(NCHUNKS - 1, 0, False)

    for b in range(2):
        c = NCHUNKS - 1 - b
        row0 = base + c * CHUNK
        pltpu.make_async_copy(
            outcs[b], outf_hbm.at[pl.ds(row0 * HIDDEN, OUT_WORDS)],
            sems[b]).wait()


@functools.partial(
    pl.kernel,
    mesh=plsc.VectorSubcoreMesh(core_axis_name="c", subcore_axis_name="s"),
    compiler_params=pltpu.CompilerParams(needs_layout_passes=False),
    out_type=jax.ShapeDtypeStruct((N * HIDDEN,), jnp.float32),
    scratch_types=[
        pltpu.VMEM((TAB_WORDS,), jnp.int32),
        pltpu.VMEM((XWIN + LANES,), jnp.int32),
        pltpu.VMEM((XWIN + LANES,), jnp.int32),
        pltpu.VMEM((OUT_WORDS,), jnp.float32),
        pltpu.VMEM((OUT_WORDS,), jnp.float32),
        pltpu.SemaphoreType.DMA,
        pltpu.SemaphoreType.DMA,
        pltpu.SemaphoreType.DMA,
        pltpu.SemaphoreType.DMA,
    ],
)
def _sc_encode(xf_hbm, tabf_hbm, outf_hbm, tab_v, xc0, xc1, outc0, outc1,
               sem0, sem1, xsem0, xsem1):
    _body(xf_hbm, tabf_hbm, outf_hbm, tab_v, xc0, xc1, outc0, outc1,
          sem0, sem1, xsem0, xsem1)


def kernel(x, emb):
    n = x.shape[0]
    tabbf = _tc_prep(emb)
    tabp = tabbf.reshape(TAB_ROWS, 4, 2, LANES).transpose(0, 1, 3, 2)
    tab_i32 = jax.lax.bitcast_convert_type(
        tabp.reshape(TAB_ROWS * HWORDS, 2), jnp.int32)
    outf = _sc_encode(x.astype(jnp.int32).reshape(-1), tab_i32)
    return outf.reshape(n, HIDDEN)
